# baseline (speedup 1.0000x reference)
import functools

import jax
import jax.numpy as jnp
from jax import lax
from jax.experimental import pallas as pl
from jax.experimental.pallas import tpu as pltpu, tpu_sc as plsc

EMB = 64
PRETRAIN = 768
PACK = 3 * EMB


def _proj_body(id_ref, que_ref, ana_ref, qW_ref, qb_ref, aW_ref, ab_ref, out_ref):
    out_ref[:, 0:EMB] = id_ref[...]
    out_ref[:, EMB:2 * EMB] = (
        jnp.dot(que_ref[...], qW_ref[...], preferred_element_type=jnp.float32)
        + qb_ref[...]
    )
    out_ref[:, 2 * EMB:3 * EMB] = (
        jnp.dot(ana_ref[...], aW_ref[...], preferred_element_type=jnp.float32)
        + ab_ref[...]
    )


def _project_pack(id_table, que_table, que_W, que_b, ana_table, ana_W, ana_b):
    rows = que_table.shape[0]
    rb = 2000
    assert rows % rb == 0
    return pl.pallas_call(
        _proj_body,
        grid=(rows // rb,),
        in_specs=[
            pl.BlockSpec((rb, EMB), lambda i: (i, 0)),
            pl.BlockSpec((rb, PRETRAIN), lambda i: (i, 0)),
            pl.BlockSpec((rb, PRETRAIN), lambda i: (i, 0)),
            pl.BlockSpec((PRETRAIN, EMB), lambda i: (0, 0)),
            pl.BlockSpec((1, EMB), lambda i: (0, 0)),
            pl.BlockSpec((PRETRAIN, EMB), lambda i: (0, 0)),
            pl.BlockSpec((1, EMB), lambda i: (0, 0)),
        ],
        out_specs=pl.BlockSpec((rb, PACK), lambda i: (i, 0)),
        out_shape=jax.ShapeDtypeStruct((rows, PACK), jnp.float32),
    )(id_table, que_table, ana_table, que_W, que_b.reshape(1, EMB),
      ana_W, ana_b.reshape(1, EMB))


def _type_body(types_ref, tt_ref, out_ref):
    tf = types_ref[...].astype(jnp.float32)
    tfe = tf[:, :, None]
    out_ref[...] = tt_ref[0, :] + tfe * (tt_ref[1, :] - tt_ref[0, :])


def _type_select(types, type_table):
    b, l = types.shape
    bm = 64
    assert b % bm == 0
    return pl.pallas_call(
        _type_body,
        grid=(b // bm,),
        in_specs=[
            pl.BlockSpec((bm, l), lambda i: (i, 0)),
            pl.BlockSpec((2, EMB), lambda i: (0, 0)),
        ],
        out_specs=pl.BlockSpec((bm, l, EMB), lambda i: (i, 0, 0)),
        out_shape=jax.ShapeDtypeStruct((b, l, EMB), jnp.float32),
    )(types, type_table)


@functools.lru_cache(maxsize=None)
def _make_gather(ntok):
    info = plsc.get_sparse_core_info()
    nc, ns = info.num_cores, info.num_subcores
    nw = nc * ns
    assert ntok % nw == 0
    per_w = ntok // nw
    chunk = 128
    assert per_w % (2 * chunk) == 0
    ngrp = per_w // (2 * chunk)

    mesh = plsc.VectorSubcoreMesh(core_axis_name="c", subcore_axis_name="s")

    @functools.partial(
        pl.kernel,
        mesh=mesh,
        compiler_params=pltpu.CompilerParams(use_tc_tiling_on_sc=False),
        out_type=[jax.ShapeDtypeStruct((ntok, EMB), jnp.float32) for _ in range(4)],
        scratch_types=[
            pltpu.VMEM((per_w,), jnp.int32),
            pltpu.VMEM((per_w,), jnp.int32),
            [pltpu.VMEM((chunk, PACK), jnp.float32) for _ in range(2)],
            [pltpu.VMEM((chunk, EMB), jnp.float32) for _ in range(2)],
            [pltpu.SemaphoreType.DMA for _ in range(2)],
            [pltpu.SemaphoreType.DMA for _ in range(2)],
        ],
    )
    def gather_k(qs_hbm, types_hbm, packed_hbm, type_tbl_hbm,
                 o_id, o_que, o_ana, o_type,
                 idx_v, tidx_v, rows, trows, sem_g, sem_s):
        wid = lax.axis_index("s") * nc + lax.axis_index("c")
        base = wid * per_w
        outs = (o_id, o_que, o_ana)

        pltpu.sync_copy(qs_hbm.at[pl.ds(base, per_w)], idx_v)
        pltpu.sync_copy(types_hbm.at[pl.ds(base, per_w)], tidx_v)

        def gathers(c, s):
            return [
                pltpu.make_async_copy(
                    packed_hbm.at[idx_v.at[pl.ds(c * chunk, chunk)]],
                    rows[s], sem_g[s]),
                pltpu.make_async_copy(
                    type_tbl_hbm.at[tidx_v.at[pl.ds(c * chunk, chunk)]],
                    trows[s], sem_g[s]),
            ]

        def scatters(c, s):
            out_slice = pl.ds(base + c * chunk, chunk)
            return [pltpu.make_async_copy(
                rows[s].at[:, pl.ds(t * EMB, EMB)],
                outs[t].at[out_slice],
                sem_s[s])
                for t in range(3)] + [
                pltpu.make_async_copy(trows[s], o_type.at[out_slice], sem_s[s])]

        def fire(cps):
            for cp in cps:
                cp.start()

        def drain(cps):
            for cp in cps:
                cp.wait()

        fire(gathers(0, 0))

        def group(g, carry):
            c0 = 2 * g
            drain(gathers(c0, 0))

            @pl.when(g > 0)
            def _():
                drain(scatters(c0 - 1, 1))

            fire(gathers(c0 + 1, 1))
            fire(scatters(c0, 0))
            drain(gathers(c0 + 1, 1))
            drain(scatters(c0, 0))

            @pl.when(g < ngrp - 1)
            def _():
                fire(gathers(c0 + 2, 0))

            fire(scatters(c0 + 1, 1))
            return carry

        lax.fori_loop(0, ngrp, group, 0)
        drain(scatters(2 * ngrp - 1, 1))

    return gather_k


def kernel(qs, types, id_table, que_table, que_W, que_b, ana_table, ana_W, ana_b, type_table):
    b, l = qs.shape
    ntok = b * l
    packed = _project_pack(id_table, que_table, que_W, que_b, ana_table, ana_W, ana_b)
    gather = _make_gather(ntok)
    o_id, o_que, o_ana, o_type = gather(
        qs.reshape(ntok), types.reshape(ntok), packed, type_table)
    return (
        o_id.reshape(b, l, EMB),
        o_que.reshape(b, l, EMB),
        o_ana.reshape(b, l, EMB),
        o_type.reshape(b, l, EMB),
    )


# speedup vs baseline: 6.9465x; 6.9465x over previous
import functools

import jax
import jax.numpy as jnp
from jax import lax
from jax.experimental import pallas as pl
from jax.experimental.pallas import tpu as pltpu, tpu_sc as plsc

EMB = 64
PRETRAIN = 768
PACK = 4 * EMB


def _proj_body(id_ref, que_ref, ana_ref, qW_ref, qb_ref, aW_ref, ab_ref,
               tt_ref, out_ref):
    m = id_ref.shape[0]
    proj_q = (
        jnp.dot(que_ref[...], qW_ref[...], preferred_element_type=jnp.float32)
        + qb_ref[...]
    )
    proj_a = (
        jnp.dot(ana_ref[...], aW_ref[...], preferred_element_type=jnp.float32)
        + ab_ref[...]
    )
    row192 = jnp.concatenate([id_ref[...], proj_q, proj_a], axis=1)
    dup = jnp.broadcast_to(row192[:, None, :], (m, 2, 3 * EMB))
    out_ref[:, 0:3 * EMB] = dup.reshape(2 * m, 3 * EMB)
    tcol = jnp.broadcast_to(tt_ref[...][None, :, :], (m, 2, EMB))
    out_ref[:, 3 * EMB:PACK] = tcol.reshape(2 * m, EMB)


def _project_pack(id_table, que_table, que_W, que_b, ana_table, ana_W, ana_b,
                  type_table):
    rows = que_table.shape[0]
    rb = 1000
    assert rows % rb == 0
    return pl.pallas_call(
        _proj_body,
        grid=(rows // rb,),
        in_specs=[
            pl.BlockSpec((rb, EMB), lambda i: (i, 0)),
            pl.BlockSpec((rb, PRETRAIN), lambda i: (i, 0)),
            pl.BlockSpec((rb, PRETRAIN), lambda i: (i, 0)),
            pl.BlockSpec((PRETRAIN, EMB), lambda i: (0, 0)),
            pl.BlockSpec((1, EMB), lambda i: (0, 0)),
            pl.BlockSpec((PRETRAIN, EMB), lambda i: (0, 0)),
            pl.BlockSpec((1, EMB), lambda i: (0, 0)),
            pl.BlockSpec((2, EMB), lambda i: (0, 0)),
        ],
        out_specs=pl.BlockSpec((2 * rb, PACK), lambda i: (i, 0)),
        out_shape=jax.ShapeDtypeStruct((2 * rows, PACK), jnp.float32),
    )(id_table, que_table, ana_table, que_W, que_b.reshape(1, EMB),
      ana_W, ana_b.reshape(1, EMB), type_table)


@functools.lru_cache(maxsize=None)
def _make_gather(ntok):
    info = plsc.get_sparse_core_info()
    nc, ns = info.num_cores, info.num_subcores
    nw = nc * ns
    assert ntok % nw == 0
    per_w = ntok // nw
    chunk = 128
    assert per_w % (2 * chunk) == 0
    ngrp = per_w // (2 * chunk)

    mesh = plsc.VectorSubcoreMesh(core_axis_name="c", subcore_axis_name="s")

    @functools.partial(
        pl.kernel,
        mesh=mesh,
        compiler_params=pltpu.CompilerParams(use_tc_tiling_on_sc=False),
        out_type=[jax.ShapeDtypeStruct((ntok, EMB), jnp.float32) for _ in range(4)],
        scratch_types=[
            pltpu.VMEM((per_w,), jnp.int32),
            [pltpu.VMEM((chunk, PACK), jnp.float32) for _ in range(2)],
            [pltpu.SemaphoreType.DMA for _ in range(2)],
            [pltpu.SemaphoreType.DMA for _ in range(2)],
        ],
    )
    def gather_k(key_hbm, packed_hbm, o_id, o_que, o_ana, o_type,
                 idx_v, rows, sem_g, sem_s):
        wid = lax.axis_index("s") * nc + lax.axis_index("c")
        base = wid * per_w
        outs = (o_id, o_que, o_ana, o_type)

        pltpu.sync_copy(key_hbm.at[pl.ds(base, per_w)], idx_v)

        def gathers(c, s):
            return [pltpu.make_async_copy(
                packed_hbm.at[idx_v.at[pl.ds(c * chunk, chunk)]],
                rows[s], sem_g[s])]

        def scatters(c, s):
            out_slice = pl.ds(base + c * chunk, chunk)
            return [pltpu.make_async_copy(
                rows[s].at[:, pl.ds(t * EMB, EMB)],
                outs[t].at[out_slice],
                sem_s[s])
                for t in range(4)]

        def fire(cps):
            for cp in cps:
                cp.start()

        def drain(cps):
            for cp in cps:
                cp.wait()

        fire(gathers(0, 0))

        def group(g, carry):
            c0 = 2 * g
            drain(gathers(c0, 0))

            @pl.when(g > 0)
            def _():
                drain(scatters(c0 - 1, 1))

            fire(gathers(c0 + 1, 1))
            fire(scatters(c0, 0))
            drain(gathers(c0 + 1, 1))
            drain(scatters(c0, 0))

            @pl.when(g < ngrp - 1)
            def _():
                fire(gathers(c0 + 2, 0))

            fire(scatters(c0 + 1, 1))
            return carry

        lax.fori_loop(0, ngrp, group, 0)
        drain(scatters(2 * ngrp - 1, 1))

    return gather_k


def kernel(qs, types, id_table, que_table, que_W, que_b, ana_table, ana_W, ana_b, type_table):
    b, l = qs.shape
    ntok = b * l
    packed = _project_pack(id_table, que_table, que_W, que_b,
                           ana_table, ana_W, ana_b, type_table)
    keys = (qs * 2 + types).reshape(ntok)
    gather = _make_gather(ntok)
    o_id, o_que, o_ana, o_type = gather(keys, packed)
    return (
        o_id.reshape(b, l, EMB),
        o_que.reshape(b, l, EMB),
        o_ana.reshape(b, l, EMB),
        o_type.reshape(b, l, EMB),
    )


# speedup vs baseline: 7.6899x; 1.1070x over previous
import functools

import jax
import jax.numpy as jnp
from jax import lax
from jax.experimental import pallas as pl
from jax.experimental.pallas import tpu as pltpu, tpu_sc as plsc

EMB = 64
PRETRAIN = 768
PACK = 3 * EMB


def _proj_body(id_ref, que_ref, ana_ref, qW_ref, qb_ref, aW_ref, ab_ref, out_ref):
    out_ref[:, 0:EMB] = id_ref[...]
    out_ref[:, EMB:2 * EMB] = (
        jnp.dot(que_ref[...], qW_ref[...], preferred_element_type=jnp.float32)
        + qb_ref[...]
    )
    out_ref[:, 2 * EMB:3 * EMB] = (
        jnp.dot(ana_ref[...], aW_ref[...], preferred_element_type=jnp.float32)
        + ab_ref[...]
    )


def _project_pack(id_table, que_table, que_W, que_b, ana_table, ana_W, ana_b):
    rows = que_table.shape[0]
    rb = 2000
    assert rows % rb == 0
    return pl.pallas_call(
        _proj_body,
        grid=(rows // rb,),
        in_specs=[
            pl.BlockSpec((rb, EMB), lambda i: (i, 0)),
            pl.BlockSpec((rb, PRETRAIN), lambda i: (i, 0)),
            pl.BlockSpec((rb, PRETRAIN), lambda i: (i, 0)),
            pl.BlockSpec((PRETRAIN, EMB), lambda i: (0, 0)),
            pl.BlockSpec((1, EMB), lambda i: (0, 0)),
            pl.BlockSpec((PRETRAIN, EMB), lambda i: (0, 0)),
            pl.BlockSpec((1, EMB), lambda i: (0, 0)),
        ],
        out_specs=pl.BlockSpec((rb, PACK), lambda i: (i, 0)),
        out_shape=jax.ShapeDtypeStruct((rows, PACK), jnp.float32),
    )(id_table, que_table, ana_table, que_W, que_b.reshape(1, EMB),
      ana_W, ana_b.reshape(1, EMB))


def _type_body(types_ref, tt_ref, out_ref):
    tf = types_ref[...].astype(jnp.float32)
    tfe = tf[:, :, None]
    out_ref[...] = tt_ref[0, :] + tfe * (tt_ref[1, :] - tt_ref[0, :])


def _type_select(types, type_table):
    b, l = types.shape
    bm = 256
    assert b % bm == 0
    return pl.pallas_call(
        _type_body,
        grid=(b // bm,),
        in_specs=[
            pl.BlockSpec((bm, l), lambda i: (i, 0)),
            pl.BlockSpec((2, EMB), lambda i: (0, 0)),
        ],
        out_specs=pl.BlockSpec((bm, l, EMB), lambda i: (i, 0, 0)),
        out_shape=jax.ShapeDtypeStruct((b, l, EMB), jnp.float32),
    )(types, type_table)


@functools.lru_cache(maxsize=None)
def _make_gather(ntok):
    info = plsc.get_sparse_core_info()
    nc, ns = info.num_cores, info.num_subcores
    nw = nc * ns
    assert ntok % nw == 0
    per_w = ntok // nw
    chunk = 128
    assert per_w % (2 * chunk) == 0
    ngrp = per_w // (2 * chunk)

    mesh = plsc.VectorSubcoreMesh(core_axis_name="c", subcore_axis_name="s")

    @functools.partial(
        pl.kernel,
        mesh=mesh,
        compiler_params=pltpu.CompilerParams(use_tc_tiling_on_sc=False),
        out_type=[jax.ShapeDtypeStruct((ntok, EMB), jnp.float32) for _ in range(3)],
        scratch_types=[
            pltpu.VMEM((per_w,), jnp.int32),
            [pltpu.VMEM((chunk, PACK), jnp.float32) for _ in range(2)],
            [pltpu.SemaphoreType.DMA for _ in range(2)],
            [pltpu.SemaphoreType.DMA for _ in range(2)],
        ],
    )
    def gather_k(qs_hbm, packed_hbm, o_id, o_que, o_ana,
                 idx_v, rows, sem_g, sem_s):
        wid = lax.axis_index("s") * nc + lax.axis_index("c")
        base = wid * per_w
        outs = (o_id, o_que, o_ana)

        pltpu.sync_copy(qs_hbm.at[pl.ds(base, per_w)], idx_v)

        def gathers(c, s):
            return [pltpu.make_async_copy(
                packed_hbm.at[idx_v.at[pl.ds(c * chunk, chunk)]],
                rows[s], sem_g[s])]

        def scatters(c, s):
            out_slice = pl.ds(base + c * chunk, chunk)
            return [pltpu.make_async_copy(
                rows[s].at[:, pl.ds(t * EMB, EMB)],
                outs[t].at[out_slice],
                sem_s[s])
                for t in range(3)]

        def fire(cps):
            for cp in cps:
                cp.start()

        def drain(cps):
            for cp in cps:
                cp.wait()

        fire(gathers(0, 0))

        def group(g, carry):
            c0 = 2 * g
            drain(gathers(c0, 0))

            @pl.when(g > 0)
            def _():
                drain(scatters(c0 - 1, 1))

            fire(gathers(c0 + 1, 1))
            fire(scatters(c0, 0))
            drain(gathers(c0 + 1, 1))
            drain(scatters(c0, 0))

            @pl.when(g < ngrp - 1)
            def _():
                fire(gathers(c0 + 2, 0))

            fire(scatters(c0 + 1, 1))
            return carry

        lax.fori_loop(0, ngrp, group, 0)
        drain(scatters(2 * ngrp - 1, 1))

    return gather_k


def kernel(qs, types, id_table, que_table, que_W, que_b, ana_table, ana_W, ana_b, type_table):
    b, l = qs.shape
    ntok = b * l
    packed = _project_pack(id_table, que_table, que_W, que_b, ana_table, ana_W, ana_b)
    o_type = _type_select(types, type_table)
    gather = _make_gather(ntok)
    o_id, o_que, o_ana = gather(qs.reshape(ntok), packed)
    return (
        o_id.reshape(b, l, EMB),
        o_que.reshape(b, l, EMB),
        o_ana.reshape(b, l, EMB),
        o_type,
    )
